# baseline (speedup 1.0000x reference)
import functools

import jax
import jax.numpy as jnp
from jax import lax
from jax.experimental import pallas as pl
from jax.experimental.pallas import tpu as pltpu
from jax.experimental.pallas import tpu_sc as plsc

HID = 128

NC = 2
NS = 16
NW = NC * NS
LANES = 16

FSPLIT = 8
ESPLIT = NW // FSPLIT
FPT = HID // FSPLIT
PAIRS = FPT // 2
STRIDE = PAIRS + 1
CHUNK = 4000
NBUF = 2


def _sc_partial_dots(zt, pos_flat, neg_flat):
  B = (pos_flat.shape[0] + neg_flat.shape[0]) // 2
  per_tile = B // ESPLIT
  n_chunks = per_tile // CHUNK
  NF = zt.shape[1] * zt.shape[2]
  mesh = plsc.VectorSubcoreMesh(core_axis_name="c", subcore_axis_name="s",
                                num_cores=NC, num_subcores=NS)

  @functools.partial(
      pl.kernel,
      out_type=jax.ShapeDtypeStruct((FSPLIT * B,), jnp.float32),
      mesh=mesh,
      compiler_params=pltpu.CompilerParams(needs_layout_passes=False,
                                           disable_bounds_checks=True),
      scratch_types=[
          pltpu.VMEM((NF,), jnp.float32),
          pltpu.VMEM((CHUNK,), jnp.int32),
          pltpu.VMEM((CHUNK,), jnp.int32),
          pltpu.VMEM((CHUNK,), jnp.int32),
          pltpu.VMEM((CHUNK,), jnp.int32),
          pltpu.VMEM((CHUNK,), jnp.float32),
          pltpu.VMEM((CHUNK,), jnp.float32),
          pltpu.SemaphoreType.DMA,
          pltpu.SemaphoreType.DMA,
          pltpu.SemaphoreType.DMA,
          pltpu.SemaphoreType.DMA,
      ],
  )
  def sc_kernel(zt_hbm, pos_hbm, neg_hbm, out_hbm, zloc,
                src0, src1, dst0, dst1, part0, part1,
                sem_i0, sem_i1, sem_w0, sem_w1):
    srcs, dsts, parts = [src0, src1], [dst0, dst1], [part0, part1]
    sem_i, sem_w = [sem_i0, sem_i1], [sem_w0, sem_w1]
    wid = lax.axis_index("s") * NC + lax.axis_index("c")
    fg = wid % FSPLIT
    q = wid // FSPLIT
    e_base = q * per_tile
    in_base = (q % 2) * per_tile
    E = B // 2
    pltpu.sync_copy(zt_hbm.at[pl.ds(fg * NF, NF)], zloc)

    def start_idx(c, b):
      off = in_base + c * CHUNK

      @pl.when(q < ESPLIT // 2)
      def _():
        pltpu.async_copy(pos_hbm.at[pl.ds(off, CHUNK)], srcs[b], sem_i[b])
        pltpu.async_copy(pos_hbm.at[pl.ds(E + off, CHUNK)], dsts[b], sem_i[b])

      @pl.when(q >= ESPLIT // 2)
      def _():
        pltpu.async_copy(neg_hbm.at[pl.ds(off, CHUNK)], srcs[b], sem_i[b])
        pltpu.async_copy(neg_hbm.at[pl.ds(E + off, CHUNK)], dsts[b], sem_i[b])

    for b in range(NBUF):
      start_idx(b, b)

    n_pairs = n_chunks // NBUF

    def pair_body(i, carry):
      for b in range(NBUF):
        c = i * NBUF + b
        off = e_base + c * CHUNK
        pltpu.make_async_copy(pos_hbm.at[pl.ds(0, CHUNK)], srcs[b],
                              sem_i[b]).wait()
        pltpu.make_async_copy(pos_hbm.at[pl.ds(0, CHUNK)], dsts[b],
                              sem_i[b]).wait()

        @pl.when(i + 1 < n_pairs)
        def _():
          start_idx(c + NBUF, b)

        @pl.when(c >= NBUF)
        def _():
          pltpu.make_async_copy(parts[b], out_hbm.at[pl.ds(0, CHUNK)],
                                sem_w[b]).wait()

        def do_group(g):
          nsrc = srcs[b][pl.ds(g * LANES, LANES)]
          ndst = dsts[b][pl.ds(g * LANES, LANES)]
          isrc = lax.shift_left(nsrc, 3) + nsrc
          idst = lax.shift_left(ndst, 3) + ndst
          acc0 = jnp.zeros((2 * LANES,), jnp.bfloat16)
          acc1 = jnp.zeros((2 * LANES,), jnp.bfloat16)
          for j in range(PAIRS):
            wa = plsc.load_gather(zloc, [isrc + j])
            wb = plsc.load_gather(zloc, [idst + j])
            pa = plsc.bitcast(wa, jnp.bfloat16)
            pb = plsc.bitcast(wb, jnp.bfloat16)
            if j % 2 == 0:
              acc0 = acc0 + pa * pb
            else:
              acc1 = acc1 + pa * pb
          acc = acc0 + acc1
          lo, hi = plsc.unpack(acc, format=plsc.PackFormat.INTERLEAVED)
          parts[b][pl.ds(g * LANES, LANES)] = lo + hi

        def group_body(g2, carry2):
          do_group(g2 * 2)
          do_group(g2 * 2 + 1)
          return carry2

        lax.fori_loop(0, CHUNK // (2 * LANES), group_body, 0)
        pltpu.async_copy(parts[b], out_hbm.at[pl.ds(fg * B + off, CHUNK)],
                         sem_w[b])
      return carry

    lax.fori_loop(0, n_pairs, pair_body, 0)
    for b in range(NBUF):
      pltpu.make_async_copy(parts[b], out_hbm.at[pl.ds(0, CHUNK)],
                            sem_w[b]).wait()

  return sc_kernel(zt.reshape(-1), pos_flat, neg_flat)


def _tc_loss_kernel(p_ref, out_ref):
  EPS = 1e-15
  j = pl.program_id(0)

  @pl.when(j == 0)
  def _():
    out_ref[0, 0] = 0.0
    out_ref[0, 1] = 0.0

  x = jnp.sum(p_ref[:, 0], axis=0)
  s = 1.0 / (1.0 + jnp.exp(-x))
  blk = jnp.where(j == 0,
                  jnp.sum(jnp.log(s + EPS)),
                  jnp.sum(jnp.log(1.0 - s + EPS)))
  idx = jnp.where(j == 0, 0, 1)
  out_ref[0, idx] += blk


def kernel(z, edge_index, neg_edge_index):
  n = z.shape[0]
  E = edge_index.shape[1]
  B = 2 * E
  pos_weight = float(n * n - 2) / 2.0
  norm = n * n / float((n * n - 2) * 2)

  pos_flat = edge_index.reshape(-1).astype(jnp.int32)
  neg_flat = neg_edge_index.reshape(-1).astype(jnp.int32)
  z_bf = z.astype(jnp.bfloat16)
  z4 = z_bf.reshape(n, FSPLIT, PAIRS, 2).transpose(1, 0, 2, 3)
  zt = jax.lax.bitcast_convert_type(z4, jnp.float32)
  zt = jnp.pad(zt, ((0, 0), (0, 0), (0, STRIDE - PAIRS)))

  parts = _sc_partial_dots(zt, pos_flat, neg_flat)

  R = E // HID
  p4d = parts.reshape(FSPLIT, 2, R, HID)
  sums = pl.pallas_call(
      _tc_loss_kernel,
      grid=(2,),
      in_specs=[pl.BlockSpec((FSPLIT, 1, R, HID), lambda j: (0, j, 0, 0))],
      out_shape=jax.ShapeDtypeStruct((1, 2), jnp.float32),
      out_specs=pl.BlockSpec(memory_space=pltpu.SMEM),
  )(p4d)

  pos_loss = -sums[0, 0] / E
  neg_loss = -sums[0, 1] / E
  return norm * (pos_loss * pos_weight + neg_loss)


# speedup vs baseline: 1.0143x; 1.0143x over previous
import functools

import jax
import jax.numpy as jnp
from jax import lax
from jax.experimental import pallas as pl
from jax.experimental.pallas import tpu as pltpu
from jax.experimental.pallas import tpu_sc as plsc

HID = 128

NC = 2
NS = 16
NW = NC * NS
LANES = 16

FSPLIT = 8
ESPLIT = NW // FSPLIT
FPT = HID // FSPLIT
PAIRS = FPT // 2
STRIDE = PAIRS + 1
CHUNK = 4000
NBUF = 2


def _sc_partial_dots(zt, pos_flat, neg_flat):
  B = (pos_flat.shape[0] + neg_flat.shape[0]) // 2
  per_tile = B // ESPLIT
  n_chunks = per_tile // CHUNK
  NF = zt.shape[1] * zt.shape[2]
  mesh = plsc.VectorSubcoreMesh(core_axis_name="c", subcore_axis_name="s",
                                num_cores=NC, num_subcores=NS)

  @functools.partial(
      pl.kernel,
      out_type=jax.ShapeDtypeStruct((FSPLIT * B,), jnp.float32),
      mesh=mesh,
      compiler_params=pltpu.CompilerParams(needs_layout_passes=False,
                                           disable_bounds_checks=True),
      scratch_types=[
          pltpu.VMEM((NF,), jnp.float32),
          pltpu.VMEM((CHUNK,), jnp.int32),
          pltpu.VMEM((CHUNK,), jnp.int32),
          pltpu.VMEM((CHUNK,), jnp.int32),
          pltpu.VMEM((CHUNK,), jnp.int32),
          pltpu.VMEM((CHUNK,), jnp.float32),
          pltpu.VMEM((CHUNK,), jnp.float32),
          pltpu.SemaphoreType.DMA,
          pltpu.SemaphoreType.DMA,
          pltpu.SemaphoreType.DMA,
          pltpu.SemaphoreType.DMA,
      ],
  )
  def sc_kernel(zt_hbm, pos_hbm, neg_hbm, out_hbm, zloc,
                src0, src1, dst0, dst1, part0, part1,
                sem_i0, sem_i1, sem_w0, sem_w1):
    srcs, dsts, parts = [src0, src1], [dst0, dst1], [part0, part1]
    sem_i, sem_w = [sem_i0, sem_i1], [sem_w0, sem_w1]
    wid = lax.axis_index("s") * NC + lax.axis_index("c")
    fg = wid % FSPLIT
    q = wid // FSPLIT
    e_base = q * per_tile
    in_base = (q % 2) * per_tile
    E = B // 2
    pltpu.sync_copy(zt_hbm.at[pl.ds(fg * NF, NF)], zloc)

    def start_idx(c, b):
      off = in_base + c * CHUNK

      @pl.when(q < ESPLIT // 2)
      def _():
        pltpu.async_copy(pos_hbm.at[pl.ds(off, CHUNK)], srcs[b], sem_i[b])
        pltpu.async_copy(pos_hbm.at[pl.ds(E + off, CHUNK)], dsts[b], sem_i[b])

      @pl.when(q >= ESPLIT // 2)
      def _():
        pltpu.async_copy(neg_hbm.at[pl.ds(off, CHUNK)], srcs[b], sem_i[b])
        pltpu.async_copy(neg_hbm.at[pl.ds(E + off, CHUNK)], dsts[b], sem_i[b])

    for b in range(NBUF):
      start_idx(b, b)

    n_pairs = n_chunks // NBUF

    def pair_body(i, carry):
      for b in range(NBUF):
        c = i * NBUF + b
        off = e_base + c * CHUNK
        pltpu.make_async_copy(pos_hbm.at[pl.ds(0, CHUNK)], srcs[b],
                              sem_i[b]).wait()
        pltpu.make_async_copy(pos_hbm.at[pl.ds(0, CHUNK)], dsts[b],
                              sem_i[b]).wait()

        @pl.when(i + 1 < n_pairs)
        def _():
          start_idx(c + NBUF, b)

        @pl.when(c >= NBUF)
        def _():
          pltpu.make_async_copy(parts[b], out_hbm.at[pl.ds(0, CHUNK)],
                                sem_w[b]).wait()

        def group_body(g, carry2):
          nsrc = srcs[b][pl.ds(g * LANES, LANES)]
          ndst = dsts[b][pl.ds(g * LANES, LANES)]
          isrc = lax.shift_left(nsrc, 3) + nsrc
          idst = lax.shift_left(ndst, 3) + ndst
          acc0 = jnp.zeros((2 * LANES,), jnp.bfloat16)
          acc1 = jnp.zeros((2 * LANES,), jnp.bfloat16)
          for j in range(PAIRS):
            wa = plsc.load_gather(zloc, [isrc + j])
            wb = plsc.load_gather(zloc, [idst + j])
            pa = plsc.bitcast(wa, jnp.bfloat16)
            pb = plsc.bitcast(wb, jnp.bfloat16)
            if j % 2 == 0:
              acc0 = acc0 + pa * pb
            else:
              acc1 = acc1 + pa * pb
          acc = acc0 + acc1
          lo, hi = plsc.unpack(acc, format=plsc.PackFormat.INTERLEAVED)
          parts[b][pl.ds(g * LANES, LANES)] = lo + hi
          return carry2

        lax.fori_loop(0, CHUNK // LANES, group_body, 0)
        pltpu.async_copy(parts[b], out_hbm.at[pl.ds(fg * B + off, CHUNK)],
                         sem_w[b])
      return carry

    lax.fori_loop(0, n_pairs, pair_body, 0)
    for b in range(NBUF):
      pltpu.make_async_copy(parts[b], out_hbm.at[pl.ds(0, CHUNK)],
                            sem_w[b]).wait()

  return sc_kernel(zt.reshape(-1), pos_flat, neg_flat)


def _tc_loss_kernel(p_ref, out_ref):
  EPS = 1e-15
  j = pl.program_id(0)

  @pl.when(j == 0)
  def _():
    out_ref[0, 0] = 0.0
    out_ref[0, 1] = 0.0

  x = jnp.sum(p_ref[:, 0], axis=0)
  s = 1.0 / (1.0 + jnp.exp(-x))
  blk = jnp.where(j == 0,
                  jnp.sum(jnp.log(s + EPS)),
                  jnp.sum(jnp.log(1.0 - s + EPS)))
  idx = jnp.where(j == 0, 0, 1)
  out_ref[0, idx] += blk


def kernel(z, edge_index, neg_edge_index):
  n = z.shape[0]
  E = edge_index.shape[1]
  B = 2 * E
  pos_weight = float(n * n - 2) / 2.0
  norm = n * n / float((n * n - 2) * 2)

  pos_flat = edge_index.reshape(-1).astype(jnp.int32)
  neg_flat = neg_edge_index.reshape(-1).astype(jnp.int32)
  z_bf = z.astype(jnp.bfloat16)
  z4 = z_bf.reshape(n, FSPLIT, PAIRS, 2).transpose(1, 0, 2, 3)
  zt = jax.lax.bitcast_convert_type(z4, jnp.float32)
  zt = jnp.pad(zt, ((0, 0), (0, 0), (0, STRIDE - PAIRS)))

  parts = _sc_partial_dots(zt, pos_flat, neg_flat)

  R = E // HID
  p4d = parts.reshape(FSPLIT, 2, R, HID)
  sums = pl.pallas_call(
      _tc_loss_kernel,
      grid=(2,),
      in_specs=[pl.BlockSpec((FSPLIT, 1, R, HID), lambda j: (0, j, 0, 0))],
      out_shape=jax.ShapeDtypeStruct((1, 2), jnp.float32),
      out_specs=pl.BlockSpec(memory_space=pltpu.SMEM),
  )(p4d)

  pos_loss = -sums[0, 0] / E
  neg_loss = -sums[0, 1] / E
  return norm * (pos_loss * pos_weight + neg_loss)
